# baseline (speedup 1.0000x reference)
import functools

import jax
import jax.numpy as jnp
from jax import lax
from jax.experimental import pallas as pl
from jax.experimental.pallas import tpu as pltpu
from jax.experimental.pallas import tpu_sc as plsc

N = 10000
E = 320000
D = 128
H = 32
_SIMILAR_WEIGHT = 1.0

NC = 2
NS = 16
L = 16
NW = NC * NS
EPW = E // NW
CH = 80
NCH = EPW // CH
NP = 10240
RPT = NP // NS

ROW_BLK = 1000


def _dyn_gather16(vec, idx16):
    return lax.gather(
        vec,
        idx16[:, None],
        dimension_numbers=lax.GatherDimensionNumbers(
            offset_dims=(), collapsed_slice_dims=(0,), start_index_map=(0,)
        ),
        slice_sizes=(1,),
        mode=lax.GatherScatterMode.PROMISE_IN_BOUNDS,
    )


def _dense_body(x_ref, wp_ref, ws_ref, b_ref, y_ref, z_ref):
    x = x_ref[...]
    y_ref[...] = jnp.dot(x, wp_ref[...], preferred_element_type=jnp.float32)
    z_ref[...] = (
        jnp.dot(x, ws_ref[...], preferred_element_type=jnp.float32) + b_ref[...]
    )


def _dense(X, W_pass, W_self, bias):
    return pl.pallas_call(
        _dense_body,
        grid=(N // ROW_BLK,),
        in_specs=[
            pl.BlockSpec((ROW_BLK, D), lambda i: (i, 0)),
            pl.BlockSpec((D, H), lambda i: (0, 0)),
            pl.BlockSpec((D, H), lambda i: (0, 0)),
            pl.BlockSpec((1, H), lambda i: (0, 0)),
        ],
        out_specs=[
            pl.BlockSpec((ROW_BLK, H), lambda i: (i, 0)),
            pl.BlockSpec((ROW_BLK, H), lambda i: (i, 0)),
        ],
        out_shape=[
            jax.ShapeDtypeStruct((N, H), jnp.float32),
            jax.ShapeDtypeStruct((N, H), jnp.float32),
        ],
    )(X, W_pass, W_self, bias)


@functools.cache
def _make_sc_segsum():
    mesh = plsc.VectorSubcoreMesh(core_axis_name="c", subcore_axis_name="s")
    return functools.partial(
        pl.kernel,
        out_type=jax.ShapeDtypeStruct((NC, NP, H), jnp.float32),
        mesh=mesh,
        compiler_params=pltpu.CompilerParams(use_tc_tiling_on_sc=False, needs_layout_passes=False),
        scratch_types=[
            pltpu.VMEM((NCH, CH), jnp.int32),
            pltpu.VMEM((NCH, CH), jnp.int32),
            pltpu.VMEM((NCH, CH), jnp.float32),
            pltpu.VMEM((CH, H), jnp.float32),
            pltpu.VMEM((RPT, H), jnp.float32),
            pltpu.VMEM_SHARED((NP, H), jnp.float32),
        ],
    )(_sc_segsum_body)


def _sc_segsum_body(y_hbm, dsti_hbm, srci_hbm, ev_hbm, out_hbm,
                    dsti_v, srci_v, ev_v, rows_v, zero_v, acc_sh):
    cid = lax.axis_index("c")
    sid = lax.axis_index("s")
    wid = sid * NC + cid

    def _zrow(i, carry):
        zero_v[i, 0:L] = jnp.zeros((L,), jnp.float32)
        zero_v[i, L:2 * L] = jnp.zeros((L,), jnp.float32)
        return carry

    lax.fori_loop(0, RPT, _zrow, 0)
    pltpu.sync_copy(zero_v, acc_sh.at[pl.ds(sid * RPT, RPT)])
    plsc.subcore_barrier()

    pltpu.sync_copy(dsti_hbm.at[wid], dsti_v)
    pltpu.sync_copy(srci_hbm.at[wid], srci_v)
    pltpu.sync_copy(ev_hbm.at[wid], ev_v)

    iota16 = lax.iota(jnp.int32, L)

    def _chunk(j, carry):
        pltpu.sync_copy(y_hbm.at[dsti_v.at[j]], rows_v)

        def _grp(g, c2):
            e16 = iota16 + g * L
            ev16 = ev_v[j, pl.ds(g * L, L)]
            for d in range(H):
                col = jnp.full((L,), d, jnp.int32)
                v = plsc.load_gather(rows_v, [e16, col])
                plsc.store_scatter(rows_v, [e16, col], v * ev16)
            return c2

        lax.fori_loop(0, CH // L, _grp, 0)
        pltpu.sync_copy(rows_v, acc_sh.at[srci_v.at[j]], add=True)
        return carry

    lax.fori_loop(0, NCH, _chunk, 0)

    plsc.subcore_barrier()
    pltpu.sync_copy(
        acc_sh.at[pl.ds(sid * RPT, RPT)],
        out_hbm.at[cid, pl.ds(sid * RPT, RPT)],
    )


def _combine_body(p_ref, z_ref, hx_ref):
    hx_ref[...] = jnp.maximum(p_ref[0] + p_ref[1] + z_ref[...], 0.0)


def _combine(partials, Z):
    return pl.pallas_call(
        _combine_body,
        grid=(N // ROW_BLK,),
        in_specs=[
            pl.BlockSpec((NC, ROW_BLK, H), lambda i: (0, i, 0)),
            pl.BlockSpec((ROW_BLK, H), lambda i: (i, 0)),
        ],
        out_specs=pl.BlockSpec((ROW_BLK, H), lambda i: (i, 0)),
        out_shape=jax.ShapeDtypeStruct((N, H), jnp.float32),
    )(partials, Z)


@functools.cache
def _make_sc_edgedot():
    mesh = plsc.VectorSubcoreMesh(core_axis_name="c", subcore_axis_name="s")
    return functools.partial(
        pl.kernel,
        out_type=jax.ShapeDtypeStruct((NW, NCH, CH), jnp.float32),
        mesh=mesh,
        compiler_params=pltpu.CompilerParams(use_tc_tiling_on_sc=False, needs_layout_passes=False),
        scratch_types=[
            pltpu.VMEM((NCH, CH), jnp.int32),
            pltpu.VMEM((NCH, CH), jnp.int32),
            pltpu.VMEM((CH, H), jnp.float32),
            pltpu.VMEM((CH, H), jnp.float32),
            pltpu.VMEM((NCH, CH), jnp.float32),
        ],
    )(_sc_edgedot_body)


def _sc_edgedot_body(hx_hbm, srci_hbm, dsti_hbm, out_hbm,
                     srci_v, dsti_v, srows_v, drows_v, dots_v):
    cid = lax.axis_index("c")
    sid = lax.axis_index("s")
    wid = sid * NC + cid

    pltpu.sync_copy(srci_hbm.at[wid], srci_v)
    pltpu.sync_copy(dsti_hbm.at[wid], dsti_v)

    iota16 = lax.iota(jnp.int32, L)

    def _chunk(j, carry):
        pltpu.sync_copy(hx_hbm.at[srci_v.at[j]], srows_v)
        pltpu.sync_copy(hx_hbm.at[dsti_v.at[j]], drows_v)

        def _grp(g, c2):
            e16 = iota16 + g * L
            acc = jnp.zeros((L,), jnp.float32)
            for d in range(H):
                col = jnp.full((L,), d, jnp.int32)
                a = plsc.load_gather(srows_v, [e16, col])
                b = plsc.load_gather(drows_v, [e16, col])
                acc = acc + a * b
            dots_v[j, pl.ds(g * L, L)] = acc
            return c2

        lax.fori_loop(0, CH // L, _grp, 0)
        return carry

    lax.fori_loop(0, NCH, _chunk, 0)
    pltpu.sync_copy(dots_v, out_hbm.at[wid])


def _loss_body(s_ref, c_ref, o_ref):
    s = s_ref[...]
    sx = 1.0 / (1.0 + jnp.exp(-s))
    sxc = jnp.clip(sx, 1e-12, 1.0 - 1e-7)
    w = jnp.where(sxc < 0.5, _SIMILAR_WEIGHT, 1.0)
    cf = c_ref[...].astype(jnp.float32)
    v = w * -(cf * jnp.log(sxc) + (1.0 - cf) * jnp.log(1.0 - sxc))
    o_ref[...] = (jnp.sum(v) / E).reshape(1, 1)


def _loss(logits2d, c2d):
    return pl.pallas_call(
        _loss_body,
        out_shape=jax.ShapeDtypeStruct((1, 1), jnp.float32),
    )(logits2d, c2d)


def kernel(X, edge_values, W_pass, b_pass, W_self, b_self, edge_index, C):
    src = edge_index[0]
    dst = edge_index[1]
    bias = (b_pass + b_self).reshape(1, H)

    Y, Z = _dense(X, W_pass, W_self, bias)

    src2 = src.reshape(NW, NCH, CH)
    dst2 = dst.reshape(NW, NCH, CH)
    ev2 = edge_values.reshape(NW, NCH, CH)

    partials = _make_sc_segsum()(Y, dst2, src2, ev2)
    Hx = _combine(partials[:, :N], Z)
    logits = _make_sc_edgedot()(Hx, src2, dst2)

    loss = _loss(logits.reshape(E // D, D), C.reshape(E // D, D))
    return loss[0, 0]


# speedup vs baseline: 4.5350x; 4.5350x over previous
import functools

import jax
import jax.numpy as jnp
from jax import lax
from jax.experimental import pallas as pl
from jax.experimental.pallas import tpu as pltpu
from jax.experimental.pallas import tpu_sc as plsc

N = 10000
E = 320000
D = 128
H = 32
_SIMILAR_WEIGHT = 1.0

NC = 2
NS = 16
L = 16
NW = NC * NS
EPW = E // NW
CH = 80
NCH = EPW // CH
NP = 10240
RPT = NP // NS

ROW_BLK = 1000


def _dyn_gather16(vec, idx16):
    return lax.gather(
        vec,
        idx16[:, None],
        dimension_numbers=lax.GatherDimensionNumbers(
            offset_dims=(), collapsed_slice_dims=(0,), start_index_map=(0,)
        ),
        slice_sizes=(1,),
        mode=lax.GatherScatterMode.PROMISE_IN_BOUNDS,
    )


def _dense_body(x_ref, wp_ref, ws_ref, b_ref, y_ref, z_ref):
    x = x_ref[...]
    y_ref[...] = jnp.dot(x, wp_ref[...], preferred_element_type=jnp.float32)
    z_ref[...] = (
        jnp.dot(x, ws_ref[...], preferred_element_type=jnp.float32) + b_ref[...]
    )


def _dense(X, W_pass, W_self, bias):
    return pl.pallas_call(
        _dense_body,
        grid=(N // ROW_BLK,),
        in_specs=[
            pl.BlockSpec((ROW_BLK, D), lambda i: (i, 0)),
            pl.BlockSpec((D, H), lambda i: (0, 0)),
            pl.BlockSpec((D, H), lambda i: (0, 0)),
            pl.BlockSpec((1, H), lambda i: (0, 0)),
        ],
        out_specs=[
            pl.BlockSpec((ROW_BLK, H), lambda i: (i, 0)),
            pl.BlockSpec((ROW_BLK, H), lambda i: (i, 0)),
        ],
        out_shape=[
            jax.ShapeDtypeStruct((N, H), jnp.float32),
            jax.ShapeDtypeStruct((N, H), jnp.float32),
        ],
    )(X, W_pass, W_self, bias)


@functools.cache
def _make_sc_segsum():
    mesh = plsc.VectorSubcoreMesh(core_axis_name="c", subcore_axis_name="s")
    return functools.partial(
        pl.kernel,
        out_type=jax.ShapeDtypeStruct((NC, NP, H), jnp.float32),
        mesh=mesh,
        compiler_params=pltpu.CompilerParams(use_tc_tiling_on_sc=False),
        scratch_types=[
            pltpu.VMEM((NCH, CH), jnp.int32),
            pltpu.VMEM((NCH, CH), jnp.int32),
            pltpu.VMEM((NCH, CH), jnp.float32),
            pltpu.VMEM((CH, H), jnp.float32),
            pltpu.VMEM((CH, H), jnp.float32),
            pltpu.VMEM((RPT, H), jnp.float32),
            pltpu.VMEM_SHARED((NP, H), jnp.float32),
            pltpu.SemaphoreType.DMA,
            pltpu.SemaphoreType.DMA,
        ],
    )(_sc_segsum_body)


def _sc_segsum_body(y_hbm, dsti_hbm, srci_hbm, ev_hbm, out_hbm,
                    dsti_v, srci_v, ev_v, rows_a, rows_b, zero_v, acc_sh,
                    sem_a, sem_b):
    cid = lax.axis_index("c")
    sid = lax.axis_index("s")
    wid = sid * NC + cid

    def _zrow(i, carry):
        zero_v[i, 0:L] = jnp.zeros((L,), jnp.float32)
        zero_v[i, L:2 * L] = jnp.zeros((L,), jnp.float32)
        return carry

    lax.fori_loop(0, RPT, _zrow, 0)
    pltpu.sync_copy(zero_v, acc_sh.at[pl.ds(sid * RPT, RPT)])
    plsc.subcore_barrier()

    pltpu.sync_copy(dsti_hbm.at[wid], dsti_v)
    pltpu.sync_copy(srci_hbm.at[wid], srci_v)
    pltpu.sync_copy(ev_hbm.at[wid], ev_v)

    def _gather(c, buf, sem):
        return pltpu.make_async_copy(y_hbm.at[dsti_v.at[c]], buf, sem)

    def _scale_and_flush(buf, j):
        def _grp(g, c2):
            ev16 = ev_v[j, pl.ds(g * L, L)]
            for k in range(L):
                i = g * L + k
                evb = _dyn_gather16(ev16, jnp.full((L,), k, jnp.int32))
                buf[i, 0:L] = buf[i, 0:L] * evb
                buf[i, L:2 * L] = buf[i, L:2 * L] * evb
            return c2

        lax.fori_loop(0, CH // L, _grp, 0)
        pltpu.sync_copy(buf, acc_sh.at[srci_v.at[j]], add=True)

    _gather(0, rows_a, sem_a).start()

    def _pipe(j2, carry):
        c0 = 2 * j2
        _gather(c0 + 1, rows_b, sem_b).start()
        _gather(c0, rows_a, sem_a).wait()
        _scale_and_flush(rows_a, c0)
        _gather(c0 + 2, rows_a, sem_a).start()
        _gather(c0 + 1, rows_b, sem_b).wait()
        _scale_and_flush(rows_b, c0 + 1)
        return carry

    lax.fori_loop(0, (NCH - 1) // 2, _pipe, 0)
    _gather(NCH - 1, rows_a, sem_a).wait()
    _scale_and_flush(rows_a, NCH - 1)

    plsc.subcore_barrier()
    pltpu.sync_copy(
        acc_sh.at[pl.ds(sid * RPT, RPT)],
        out_hbm.at[cid, pl.ds(sid * RPT, RPT)],
    )


def _combine_body(p_ref, z_ref, hx_ref):
    hx_ref[...] = jnp.maximum(p_ref[0] + p_ref[1] + z_ref[...], 0.0)


def _combine(partials, Z):
    return pl.pallas_call(
        _combine_body,
        grid=(N // ROW_BLK,),
        in_specs=[
            pl.BlockSpec((NC, ROW_BLK, H), lambda i: (0, i, 0)),
            pl.BlockSpec((ROW_BLK, H), lambda i: (i, 0)),
        ],
        out_specs=pl.BlockSpec((ROW_BLK, H), lambda i: (i, 0)),
        out_shape=jax.ShapeDtypeStruct((N, H), jnp.float32),
    )(partials, Z)


@functools.cache
def _make_sc_edgedot():
    mesh = plsc.VectorSubcoreMesh(core_axis_name="c", subcore_axis_name="s")
    return functools.partial(
        pl.kernel,
        out_type=jax.ShapeDtypeStruct((NW, NCH, CH), jnp.float32),
        mesh=mesh,
        compiler_params=pltpu.CompilerParams(use_tc_tiling_on_sc=False),
        scratch_types=[
            pltpu.VMEM((NCH, CH), jnp.int32),
            pltpu.VMEM((NCH, CH), jnp.int32),
            pltpu.VMEM((CH, H), jnp.float32),
            pltpu.VMEM((CH, H), jnp.float32),
            pltpu.VMEM((CH, H), jnp.float32),
            pltpu.VMEM((CH, H), jnp.float32),
            pltpu.VMEM((NCH, CH), jnp.float32),
            pltpu.SemaphoreType.DMA,
            pltpu.SemaphoreType.DMA,
        ],
    )(_sc_edgedot_body)


def _sc_edgedot_body(hx_hbm, srci_hbm, dsti_hbm, out_hbm,
                     srci_v, dsti_v, sr_a, dr_a, sr_b, dr_b, dots_v,
                     sem_a, sem_b):
    cid = lax.axis_index("c")
    sid = lax.axis_index("s")
    wid = sid * NC + cid

    pltpu.sync_copy(srci_hbm.at[wid], srci_v)
    pltpu.sync_copy(dsti_hbm.at[wid], dsti_v)

    iota16 = lax.iota(jnp.int32, L)

    def _g_src(c, buf, sem):
        return pltpu.make_async_copy(hx_hbm.at[srci_v.at[c]], buf, sem)

    def _g_dst(c, buf, sem):
        return pltpu.make_async_copy(hx_hbm.at[dsti_v.at[c]], buf, sem)

    def _start(c, sbuf, dbuf, sem):
        _g_src(c, sbuf, sem).start()
        _g_dst(c, dbuf, sem).start()

    def _wait(c, sbuf, dbuf, sem):
        _g_src(c, sbuf, sem).wait()
        _g_dst(c, dbuf, sem).wait()

    def _dot(sbuf, dbuf, j):
        def _grp(g, c2):
            acc = jnp.zeros((L,), jnp.float32)
            for k in range(L):
                i = g * L + k
                v = (sbuf[i, 0:L] * dbuf[i, 0:L]
                     + sbuf[i, L:2 * L] * dbuf[i, L:2 * L])
                for sh in (1, 2, 4, 8):
                    v = v + _dyn_gather16(v, jnp.bitwise_xor(iota16, sh))
                acc = jnp.where(iota16 == k, v, acc)
            dots_v[j, pl.ds(g * L, L)] = acc
            return c2

        lax.fori_loop(0, CH // L, _grp, 0)

    _start(0, sr_a, dr_a, sem_a)

    def _pipe(j2, carry):
        c0 = 2 * j2
        _start(c0 + 1, sr_b, dr_b, sem_b)
        _wait(c0, sr_a, dr_a, sem_a)
        _dot(sr_a, dr_a, c0)
        _start(c0 + 2, sr_a, dr_a, sem_a)
        _wait(c0 + 1, sr_b, dr_b, sem_b)
        _dot(sr_b, dr_b, c0 + 1)
        return carry

    lax.fori_loop(0, (NCH - 1) // 2, _pipe, 0)
    _wait(NCH - 1, sr_a, dr_a, sem_a)
    _dot(sr_a, dr_a, NCH - 1)

    pltpu.sync_copy(dots_v, out_hbm.at[wid])


def _loss_body(s_ref, c_ref, o_ref):
    s = s_ref[...]
    sx = 1.0 / (1.0 + jnp.exp(-s))
    sxc = jnp.clip(sx, 1e-12, 1.0 - 1e-7)
    w = jnp.where(sxc < 0.5, _SIMILAR_WEIGHT, 1.0)
    cf = c_ref[...].astype(jnp.float32)
    v = w * -(cf * jnp.log(sxc) + (1.0 - cf) * jnp.log(1.0 - sxc))
    o_ref[...] = (jnp.sum(v) / E).reshape(1, 1)


def _loss(logits2d, c2d):
    return pl.pallas_call(
        _loss_body,
        out_shape=jax.ShapeDtypeStruct((1, 1), jnp.float32),
    )(logits2d, c2d)


def kernel(X, edge_values, W_pass, b_pass, W_self, b_self, edge_index, C):
    src = edge_index[0]
    dst = edge_index[1]
    bias = (b_pass + b_self).reshape(1, H)

    Y, Z = _dense(X, W_pass, W_self, bias)

    src2 = src.reshape(NW, NCH, CH)
    dst2 = dst.reshape(NW, NCH, CH)
    ev2 = edge_values.reshape(NW, NCH, CH)

    partials = _make_sc_segsum()(Y, dst2, src2, ev2)
    Hx = _combine(partials[:, :N], Z)
    logits = _make_sc_edgedot()(Hx, src2, dst2)

    loss = _loss(logits.reshape(E // D, D), C.reshape(E // D, D))
    return loss[0, 0]


# speedup vs baseline: 5.8840x; 1.2975x over previous
import functools

import jax
import jax.numpy as jnp
from jax import lax
from jax.experimental import pallas as pl
from jax.experimental.pallas import tpu as pltpu
from jax.experimental.pallas import tpu_sc as plsc

N = 10000
E = 320000
D = 128
H = 32
_SIMILAR_WEIGHT = 1.0

NC = 2
NS = 16
L = 16
NW = NC * NS
EPW = E // NW
CH = 80
NCH = EPW // CH
NP = 10240
RPT = NP // NS

ROW_BLK = 1000


def _dyn_gather16(vec, idx16):
    return lax.gather(
        vec,
        idx16[:, None],
        dimension_numbers=lax.GatherDimensionNumbers(
            offset_dims=(), collapsed_slice_dims=(0,), start_index_map=(0,)
        ),
        slice_sizes=(1,),
        mode=lax.GatherScatterMode.PROMISE_IN_BOUNDS,
    )


def _dense_body(x_ref, wp_ref, ws_ref, b_ref, y_ref, z_ref):
    x = x_ref[...]
    y_ref[...] = jnp.dot(x, wp_ref[...], preferred_element_type=jnp.float32)
    z_ref[...] = (
        jnp.dot(x, ws_ref[...], preferred_element_type=jnp.float32) + b_ref[...]
    )


def _dense(X, W_pass, W_self, bias):
    return pl.pallas_call(
        _dense_body,
        grid=(N // ROW_BLK,),
        in_specs=[
            pl.BlockSpec((ROW_BLK, D), lambda i: (i, 0)),
            pl.BlockSpec((D, H), lambda i: (0, 0)),
            pl.BlockSpec((D, H), lambda i: (0, 0)),
            pl.BlockSpec((1, H), lambda i: (0, 0)),
        ],
        out_specs=[
            pl.BlockSpec((ROW_BLK, H), lambda i: (i, 0)),
            pl.BlockSpec((ROW_BLK, H), lambda i: (i, 0)),
        ],
        out_shape=[
            jax.ShapeDtypeStruct((NP, H), jnp.float32),
            jax.ShapeDtypeStruct((N, H), jnp.float32),
        ],
    )(X, W_pass, W_self, bias)


@functools.cache
def _make_sc_segsum():
    mesh = plsc.VectorSubcoreMesh(core_axis_name="c", subcore_axis_name="s")
    return functools.partial(
        pl.kernel,
        out_type=jax.ShapeDtypeStruct((NC, NP, H), jnp.float32),
        mesh=mesh,
        compiler_params=pltpu.CompilerParams(use_tc_tiling_on_sc=False),
        scratch_types=[
            pltpu.VMEM((NCH, CH), jnp.int32),
            pltpu.VMEM((NCH, CH), jnp.int32),
            pltpu.VMEM((NCH, CH), jnp.float32),
            pltpu.VMEM((CH, H), jnp.float32),
            pltpu.VMEM((CH, H), jnp.float32),
            pltpu.VMEM((RPT, H), jnp.float32),
            pltpu.VMEM_SHARED((NP, H), jnp.float32),
            pltpu.VMEM_SHARED((NP, H), jnp.float32),
            pltpu.SemaphoreType.DMA,
            pltpu.SemaphoreType.DMA,
        ],
    )(_sc_segsum_body)


def _sc_segsum_body(y_hbm, dsti_hbm, srci_hbm, ev_hbm, out_hbm,
                    dsti_v, srci_v, ev_v, rows_a, rows_b, zero_v, acc_sh,
                    y_sh, sem_a, sem_b):
    cid = lax.axis_index("c")
    sid = lax.axis_index("s")
    wid = sid * NC + cid

    def _zrow(i, carry):
        zero_v[i, 0:L] = jnp.zeros((L,), jnp.float32)
        zero_v[i, L:2 * L] = jnp.zeros((L,), jnp.float32)
        return carry

    lax.fori_loop(0, RPT, _zrow, 0)
    pltpu.sync_copy(zero_v, acc_sh.at[pl.ds(sid * RPT, RPT)])
    pltpu.sync_copy(y_hbm.at[pl.ds(sid * RPT, RPT)],
                    y_sh.at[pl.ds(sid * RPT, RPT)])
    plsc.subcore_barrier()

    pltpu.sync_copy(dsti_hbm.at[wid], dsti_v)
    pltpu.sync_copy(srci_hbm.at[wid], srci_v)
    pltpu.sync_copy(ev_hbm.at[wid], ev_v)

    def _gather(c, buf, sem):
        return pltpu.make_async_copy(y_sh.at[dsti_v.at[c]], buf, sem)

    def _scale_and_flush(buf, j):
        def _grp(g, c2):
            ev16 = ev_v[j, pl.ds(g * L, L)]
            for k in range(L):
                i = g * L + k
                evb = _dyn_gather16(ev16, jnp.full((L,), k, jnp.int32))
                buf[i, 0:L] = buf[i, 0:L] * evb
                buf[i, L:2 * L] = buf[i, L:2 * L] * evb
            return c2

        lax.fori_loop(0, CH // L, _grp, 0)
        pltpu.sync_copy(buf, acc_sh.at[srci_v.at[j]], add=True)

    _gather(0, rows_a, sem_a).start()

    def _pipe(j2, carry):
        c0 = 2 * j2
        _gather(c0 + 1, rows_b, sem_b).start()
        _gather(c0, rows_a, sem_a).wait()
        _scale_and_flush(rows_a, c0)
        _gather(c0 + 2, rows_a, sem_a).start()
        _gather(c0 + 1, rows_b, sem_b).wait()
        _scale_and_flush(rows_b, c0 + 1)
        return carry

    lax.fori_loop(0, (NCH - 1) // 2, _pipe, 0)
    _gather(NCH - 1, rows_a, sem_a).wait()
    _scale_and_flush(rows_a, NCH - 1)

    plsc.subcore_barrier()
    pltpu.sync_copy(
        acc_sh.at[pl.ds(sid * RPT, RPT)],
        out_hbm.at[cid, pl.ds(sid * RPT, RPT)],
    )


def _combine_body(p_ref, z_ref, hx_ref):
    hx_ref[...] = jnp.maximum(p_ref[0] + p_ref[1] + z_ref[...], 0.0)


def _combine(partials, Z):
    return pl.pallas_call(
        _combine_body,
        grid=(N // ROW_BLK,),
        in_specs=[
            pl.BlockSpec((NC, ROW_BLK, H), lambda i: (0, i, 0)),
            pl.BlockSpec((ROW_BLK, H), lambda i: (i, 0)),
        ],
        out_specs=pl.BlockSpec((ROW_BLK, H), lambda i: (i, 0)),
        out_shape=jax.ShapeDtypeStruct((NP, H), jnp.float32),
    )(partials, Z)


@functools.cache
def _make_sc_edgedot():
    mesh = plsc.VectorSubcoreMesh(core_axis_name="c", subcore_axis_name="s")
    return functools.partial(
        pl.kernel,
        out_type=jax.ShapeDtypeStruct((NW, NCH, CH), jnp.float32),
        mesh=mesh,
        compiler_params=pltpu.CompilerParams(use_tc_tiling_on_sc=False),
        scratch_types=[
            pltpu.VMEM((NCH, CH), jnp.int32),
            pltpu.VMEM((NCH, CH), jnp.int32),
            pltpu.VMEM((CH, H), jnp.float32),
            pltpu.VMEM((CH, H), jnp.float32),
            pltpu.VMEM((CH, H), jnp.float32),
            pltpu.VMEM((CH, H), jnp.float32),
            pltpu.VMEM((NCH, CH), jnp.float32),
            pltpu.VMEM_SHARED((NP, H), jnp.float32),
            pltpu.SemaphoreType.DMA,
            pltpu.SemaphoreType.DMA,
        ],
    )(_sc_edgedot_body)


def _sc_edgedot_body(hx_hbm, srci_hbm, dsti_hbm, out_hbm,
                     srci_v, dsti_v, sr_a, dr_a, sr_b, dr_b, dots_v,
                     hx_sh, sem_a, sem_b):
    cid = lax.axis_index("c")
    sid = lax.axis_index("s")
    wid = sid * NC + cid

    pltpu.sync_copy(hx_hbm.at[pl.ds(sid * RPT, RPT)],
                    hx_sh.at[pl.ds(sid * RPT, RPT)])
    pltpu.sync_copy(srci_hbm.at[wid], srci_v)
    pltpu.sync_copy(dsti_hbm.at[wid], dsti_v)
    plsc.subcore_barrier()

    iota16 = lax.iota(jnp.int32, L)

    def _g_src(c, buf, sem):
        return pltpu.make_async_copy(hx_sh.at[srci_v.at[c]], buf, sem)

    def _g_dst(c, buf, sem):
        return pltpu.make_async_copy(hx_sh.at[dsti_v.at[c]], buf, sem)

    def _start(c, sbuf, dbuf, sem):
        _g_src(c, sbuf, sem).start()
        _g_dst(c, dbuf, sem).start()

    def _wait(c, sbuf, dbuf, sem):
        _g_src(c, sbuf, sem).wait()
        _g_dst(c, dbuf, sem).wait()

    def _dot(sbuf, dbuf, j):
        def _grp(g, c2):
            acc = jnp.zeros((L,), jnp.float32)
            for k in range(L):
                i = g * L + k
                v = (sbuf[i, 0:L] * dbuf[i, 0:L]
                     + sbuf[i, L:2 * L] * dbuf[i, L:2 * L])
                for sh in (1, 2, 4, 8):
                    v = v + _dyn_gather16(v, jnp.bitwise_xor(iota16, sh))
                acc = jnp.where(iota16 == k, v, acc)
            dots_v[j, pl.ds(g * L, L)] = acc
            return c2

        lax.fori_loop(0, CH // L, _grp, 0)

    _start(0, sr_a, dr_a, sem_a)

    def _pipe(j2, carry):
        c0 = 2 * j2
        _start(c0 + 1, sr_b, dr_b, sem_b)
        _wait(c0, sr_a, dr_a, sem_a)
        _dot(sr_a, dr_a, c0)
        _start(c0 + 2, sr_a, dr_a, sem_a)
        _wait(c0 + 1, sr_b, dr_b, sem_b)
        _dot(sr_b, dr_b, c0 + 1)
        return carry

    lax.fori_loop(0, (NCH - 1) // 2, _pipe, 0)
    _wait(NCH - 1, sr_a, dr_a, sem_a)
    _dot(sr_a, dr_a, NCH - 1)

    pltpu.sync_copy(dots_v, out_hbm.at[wid])


def _loss_body(s_ref, c_ref, o_ref):
    s = s_ref[...]
    sx = 1.0 / (1.0 + jnp.exp(-s))
    sxc = jnp.clip(sx, 1e-12, 1.0 - 1e-7)
    w = jnp.where(sxc < 0.5, _SIMILAR_WEIGHT, 1.0)
    cf = c_ref[...].astype(jnp.float32)
    v = w * -(cf * jnp.log(sxc) + (1.0 - cf) * jnp.log(1.0 - sxc))
    o_ref[...] = (jnp.sum(v) / E).reshape(1, 1)


def _loss(logits2d, c2d):
    return pl.pallas_call(
        _loss_body,
        out_shape=jax.ShapeDtypeStruct((1, 1), jnp.float32),
    )(logits2d, c2d)


def kernel(X, edge_values, W_pass, b_pass, W_self, b_self, edge_index, C):
    src = edge_index[0]
    dst = edge_index[1]
    bias = (b_pass + b_self).reshape(1, H)

    Y, Z = _dense(X, W_pass, W_self, bias)

    src2 = src.reshape(NW, NCH, CH)
    dst2 = dst.reshape(NW, NCH, CH)
    ev2 = edge_values.reshape(NW, NCH, CH)

    partials = _make_sc_segsum()(Y, dst2, src2, ev2)
    Hx = _combine(partials[:, :N], Z)
    logits = _make_sc_edgedot()(Hx, src2, dst2)

    loss = _loss(logits.reshape(E // D, D), C.reshape(E // D, D))
    return loss[0, 0]


# speedup vs baseline: 6.6418x; 1.1288x over previous
import functools

import jax
import jax.numpy as jnp
from jax import lax
from jax.experimental import pallas as pl
from jax.experimental.pallas import tpu as pltpu
from jax.experimental.pallas import tpu_sc as plsc

N = 10000
E = 320000
D = 128
H = 32
_SIMILAR_WEIGHT = 1.0

NC = 2
NS = 16
L = 16
NW = NC * NS
EPW = E // NW
CH = 80
NCH = EPW // CH
NP = 10240
RPT = NP // NS

ROW_BLK = 1000


def _dyn_gather16(vec, idx16):
    return lax.gather(
        vec,
        idx16[:, None],
        dimension_numbers=lax.GatherDimensionNumbers(
            offset_dims=(), collapsed_slice_dims=(0,), start_index_map=(0,)
        ),
        slice_sizes=(1,),
        mode=lax.GatherScatterMode.PROMISE_IN_BOUNDS,
    )


def _dense_body(x_ref, wp_ref, ws_ref, b_ref, y_ref, z_ref):
    x = x_ref[...]
    y_ref[...] = jnp.dot(x, wp_ref[...], preferred_element_type=jnp.float32)
    z_ref[...] = (
        jnp.dot(x, ws_ref[...], preferred_element_type=jnp.float32) + b_ref[...]
    )


def _dense(X, W_pass, W_self, bias):
    return pl.pallas_call(
        _dense_body,
        grid=(N // ROW_BLK,),
        in_specs=[
            pl.BlockSpec((ROW_BLK, D), lambda i: (i, 0)),
            pl.BlockSpec((D, H), lambda i: (0, 0)),
            pl.BlockSpec((D, H), lambda i: (0, 0)),
            pl.BlockSpec((1, H), lambda i: (0, 0)),
        ],
        out_specs=[
            pl.BlockSpec((ROW_BLK, H), lambda i: (i, 0)),
            pl.BlockSpec((ROW_BLK, H), lambda i: (i, 0)),
        ],
        out_shape=[
            jax.ShapeDtypeStruct((NP, H), jnp.float32),
            jax.ShapeDtypeStruct((NP, H), jnp.float32),
        ],
    )(X, W_pass, W_self, bias)


@functools.cache
def _make_sc_segsum():
    mesh = plsc.VectorSubcoreMesh(core_axis_name="c", subcore_axis_name="s")
    return functools.partial(
        pl.kernel,
        out_type=jax.ShapeDtypeStruct((NC, NP, H), jnp.float32),
        mesh=mesh,
        compiler_params=pltpu.CompilerParams(use_tc_tiling_on_sc=False),
        scratch_types=[
            pltpu.VMEM((NCH, CH), jnp.int32),
            pltpu.VMEM((NCH, CH), jnp.int32),
            pltpu.VMEM((NCH, CH), jnp.float32),
            pltpu.VMEM((CH, H), jnp.float32),
            pltpu.VMEM((CH, H), jnp.float32),
            pltpu.VMEM((RPT, H), jnp.float32),
            pltpu.VMEM_SHARED((NP, H), jnp.float32),
            pltpu.VMEM_SHARED((NP, H), jnp.float32),
            pltpu.SemaphoreType.DMA,
            pltpu.SemaphoreType.DMA,
        ],
    )(_sc_segsum_body)


def _sc_segsum_body(y_hbm, dsti_hbm, srci_hbm, ev_hbm, out_hbm,
                    dsti_v, srci_v, ev_v, rows_a, rows_b, zero_v, acc_sh,
                    y_sh, sem_a, sem_b):
    cid = lax.axis_index("c")
    sid = lax.axis_index("s")
    wid = sid * NC + cid

    def _zrow(i, carry):
        zero_v[i, 0:L] = jnp.zeros((L,), jnp.float32)
        zero_v[i, L:2 * L] = jnp.zeros((L,), jnp.float32)
        return carry

    lax.fori_loop(0, RPT, _zrow, 0)
    pltpu.sync_copy(zero_v, acc_sh.at[pl.ds(sid * RPT, RPT)])
    pltpu.sync_copy(y_hbm.at[pl.ds(sid * RPT, RPT)],
                    y_sh.at[pl.ds(sid * RPT, RPT)])
    plsc.subcore_barrier()

    pltpu.sync_copy(dsti_hbm.at[wid], dsti_v)
    pltpu.sync_copy(srci_hbm.at[wid], srci_v)
    pltpu.sync_copy(ev_hbm.at[wid], ev_v)

    def _gather(c, buf, sem):
        return pltpu.make_async_copy(y_sh.at[dsti_v.at[c]], buf, sem)

    def _scale_and_flush(buf, j):
        def _grp(g, c2):
            ev16 = ev_v[j, pl.ds(g * L, L)]
            for k in range(L):
                i = g * L + k
                evb = _dyn_gather16(ev16, jnp.full((L,), k, jnp.int32))
                buf[i, 0:L] = buf[i, 0:L] * evb
                buf[i, L:2 * L] = buf[i, L:2 * L] * evb
            return c2

        lax.fori_loop(0, CH // L, _grp, 0)
        pltpu.sync_copy(buf, acc_sh.at[srci_v.at[j]], add=True)

    _gather(0, rows_a, sem_a).start()

    def _pipe(j2, carry):
        c0 = 2 * j2
        _gather(c0 + 1, rows_b, sem_b).start()
        _gather(c0, rows_a, sem_a).wait()
        _scale_and_flush(rows_a, c0)
        _gather(c0 + 2, rows_a, sem_a).start()
        _gather(c0 + 1, rows_b, sem_b).wait()
        _scale_and_flush(rows_b, c0 + 1)
        return carry

    lax.fori_loop(0, (NCH - 1) // 2, _pipe, 0)
    _gather(NCH - 1, rows_a, sem_a).wait()
    _scale_and_flush(rows_a, NCH - 1)

    plsc.subcore_barrier()
    pltpu.sync_copy(
        acc_sh.at[pl.ds(sid * RPT, RPT)],
        out_hbm.at[cid, pl.ds(sid * RPT, RPT)],
    )


@functools.cache
def _make_sc_edgedot():
    mesh = plsc.VectorSubcoreMesh(core_axis_name="c", subcore_axis_name="s")
    return functools.partial(
        pl.kernel,
        out_type=jax.ShapeDtypeStruct((NW, NCH, CH), jnp.float32),
        mesh=mesh,
        compiler_params=pltpu.CompilerParams(use_tc_tiling_on_sc=False),
        scratch_types=[
            pltpu.VMEM((NCH, CH), jnp.int32),
            pltpu.VMEM((NCH, CH), jnp.int32),
            pltpu.VMEM((CH, H), jnp.float32),
            pltpu.VMEM((CH, H), jnp.float32),
            pltpu.VMEM((CH, H), jnp.float32),
            pltpu.VMEM((CH, H), jnp.float32),
            pltpu.VMEM((NCH, CH), jnp.float32),
            pltpu.VMEM((RPT, H), jnp.float32),
            pltpu.VMEM((RPT, H), jnp.float32),
            pltpu.VMEM((RPT, H), jnp.float32),
            pltpu.VMEM_SHARED((NP, H), jnp.float32),
            pltpu.SemaphoreType.DMA,
            pltpu.SemaphoreType.DMA,
        ],
    )(_sc_edgedot_body)


def _sc_edgedot_body(p_hbm, z_hbm, srci_hbm, dsti_hbm, out_hbm,
                     srci_v, dsti_v, sr_a, dr_a, sr_b, dr_b, dots_v,
                     s0_v, s1_v, sz_v, hx_sh, sem_a, sem_b):
    cid = lax.axis_index("c")
    sid = lax.axis_index("s")
    wid = sid * NC + cid

    pltpu.sync_copy(p_hbm.at[0, pl.ds(sid * RPT, RPT)], s0_v)
    pltpu.sync_copy(p_hbm.at[1, pl.ds(sid * RPT, RPT)], s1_v)
    pltpu.sync_copy(z_hbm.at[pl.ds(sid * RPT, RPT)], sz_v)
    pltpu.sync_copy(srci_hbm.at[wid], srci_v)
    pltpu.sync_copy(dsti_hbm.at[wid], dsti_v)

    def _hxrow(r, carry):
        for h in range(2):
            sl = pl.ds(h * L, L)
            s0_v[r, sl] = jnp.maximum(
                s0_v[r, sl] + s1_v[r, sl] + sz_v[r, sl], 0.0
            )
        return carry

    lax.fori_loop(0, RPT, _hxrow, 0)
    pltpu.sync_copy(s0_v, hx_sh.at[pl.ds(sid * RPT, RPT)])
    plsc.subcore_barrier()

    iota16 = lax.iota(jnp.int32, L)

    def _g_src(c, buf, sem):
        return pltpu.make_async_copy(hx_sh.at[srci_v.at[c]], buf, sem)

    def _g_dst(c, buf, sem):
        return pltpu.make_async_copy(hx_sh.at[dsti_v.at[c]], buf, sem)

    def _start(c, sbuf, dbuf, sem):
        _g_src(c, sbuf, sem).start()
        _g_dst(c, dbuf, sem).start()

    def _wait(c, sbuf, dbuf, sem):
        _g_src(c, sbuf, sem).wait()
        _g_dst(c, dbuf, sem).wait()

    def _dot(sbuf, dbuf, j):
        def _grp(g, c2):
            acc = jnp.zeros((L,), jnp.float32)
            for k in range(L):
                i = g * L + k
                v = (sbuf[i, 0:L] * dbuf[i, 0:L]
                     + sbuf[i, L:2 * L] * dbuf[i, L:2 * L])
                for sh in (1, 2, 4, 8):
                    v = v + _dyn_gather16(v, jnp.bitwise_xor(iota16, sh))
                acc = jnp.where(iota16 == k, v, acc)
            dots_v[j, pl.ds(g * L, L)] = acc
            return c2

        lax.fori_loop(0, CH // L, _grp, 0)

    _start(0, sr_a, dr_a, sem_a)

    def _pipe(j2, carry):
        c0 = 2 * j2
        _start(c0 + 1, sr_b, dr_b, sem_b)
        _wait(c0, sr_a, dr_a, sem_a)
        _dot(sr_a, dr_a, c0)
        _start(c0 + 2, sr_a, dr_a, sem_a)
        _wait(c0 + 1, sr_b, dr_b, sem_b)
        _dot(sr_b, dr_b, c0 + 1)
        return carry

    lax.fori_loop(0, (NCH - 1) // 2, _pipe, 0)
    _wait(NCH - 1, sr_a, dr_a, sem_a)
    _dot(sr_a, dr_a, NCH - 1)

    pltpu.sync_copy(dots_v, out_hbm.at[wid])


def _loss_body(s_ref, c_ref, o_ref):
    s = s_ref[...]
    sx = 1.0 / (1.0 + jnp.exp(-s))
    sxc = jnp.clip(sx, 1e-12, 1.0 - 1e-7)
    w = jnp.where(sxc < 0.5, _SIMILAR_WEIGHT, 1.0)
    cf = c_ref[...].astype(jnp.float32)
    v = w * -(cf * jnp.log(sxc) + (1.0 - cf) * jnp.log(1.0 - sxc))
    o_ref[...] = (jnp.sum(v) / E).reshape(1, 1)


def _loss(logits2d, c2d):
    return pl.pallas_call(
        _loss_body,
        out_shape=jax.ShapeDtypeStruct((1, 1), jnp.float32),
    )(logits2d, c2d)


def kernel(X, edge_values, W_pass, b_pass, W_self, b_self, edge_index, C):
    src = edge_index[0]
    dst = edge_index[1]
    bias = (b_pass + b_self).reshape(1, H)

    Y, Z = _dense(X, W_pass, W_self, bias)

    src2 = src.reshape(NW, NCH, CH)
    dst2 = dst.reshape(NW, NCH, CH)
    ev2 = edge_values.reshape(NW, NCH, CH)

    partials = _make_sc_segsum()(Y, dst2, src2, ev2)
    logits = _make_sc_edgedot()(partials, Z, src2, dst2)

    loss = _loss(logits.reshape(E // D, D), C.reshape(E // D, D))
    return loss[0, 0]


# speedup vs baseline: 6.9660x; 1.0488x over previous
import functools

import jax
import jax.numpy as jnp
from jax import lax
from jax.experimental import pallas as pl
from jax.experimental.pallas import tpu as pltpu
from jax.experimental.pallas import tpu_sc as plsc

N = 10000
E = 320000
D = 128
H = 32
_SIMILAR_WEIGHT = 1.0

NC = 2
NS = 16
L = 16
NW = NC * NS
EPW = E // NW
CH = 80
NCH = EPW // CH
SCH = 5
NG = NCH // SCH
EC = SCH * CH
NP = 10240
RPT = NP // NS

ROW_BLK = 1000


def _dyn_gather16(vec, idx16):
    return lax.gather(
        vec,
        idx16[:, None],
        dimension_numbers=lax.GatherDimensionNumbers(
            offset_dims=(), collapsed_slice_dims=(0,), start_index_map=(0,)
        ),
        slice_sizes=(1,),
        mode=lax.GatherScatterMode.PROMISE_IN_BOUNDS,
    )


def _dense_body(x_ref, wp_ref, ws_ref, b_ref, y_ref, z_ref):
    x = x_ref[...]
    y_ref[...] = jnp.dot(x, wp_ref[...], preferred_element_type=jnp.float32)
    z_ref[...] = (
        jnp.dot(x, ws_ref[...], preferred_element_type=jnp.float32) + b_ref[...]
    )


def _dense(X, W_pass, W_self, bias):
    return pl.pallas_call(
        _dense_body,
        grid=(N // ROW_BLK,),
        in_specs=[
            pl.BlockSpec((ROW_BLK, D), lambda i: (i, 0)),
            pl.BlockSpec((D, H), lambda i: (0, 0)),
            pl.BlockSpec((D, H), lambda i: (0, 0)),
            pl.BlockSpec((1, H), lambda i: (0, 0)),
        ],
        out_specs=[
            pl.BlockSpec((ROW_BLK, H), lambda i: (i, 0)),
            pl.BlockSpec((ROW_BLK, H), lambda i: (i, 0)),
        ],
        out_shape=[
            jax.ShapeDtypeStruct((NP, H), jnp.float32),
            jax.ShapeDtypeStruct((NP, H), jnp.float32),
        ],
    )(X, W_pass, W_self, bias)


@functools.cache
def _make_sc_segsum():
    mesh = plsc.VectorSubcoreMesh(core_axis_name="c", subcore_axis_name="s")
    return functools.partial(
        pl.kernel,
        out_type=jax.ShapeDtypeStruct((NC, NP, H), jnp.float32),
        mesh=mesh,
        compiler_params=pltpu.CompilerParams(use_tc_tiling_on_sc=False),
        scratch_types=[
            pltpu.VMEM((NG, EC), jnp.int32),
            pltpu.VMEM((NG, EC), jnp.int32),
            pltpu.VMEM((NG, EC), jnp.float32),
            pltpu.VMEM((EC, H), jnp.float32),
            pltpu.VMEM((EC, H), jnp.float32),
            pltpu.VMEM((EC, H), jnp.float32),
            pltpu.VMEM_SHARED((NP, H), jnp.float32),
            pltpu.VMEM_SHARED((NP, H), jnp.float32),
            pltpu.SemaphoreType.DMA,
            pltpu.SemaphoreType.DMA,
            pltpu.SemaphoreType.DMA,
            pltpu.SemaphoreType.DMA,
            pltpu.SemaphoreType.DMA,
            pltpu.SemaphoreType.DMA,
        ],
    )(_sc_segsum_body)


def _sc_segsum_body(y_hbm, dsti_hbm, srci_hbm, ev_hbm, out_hbm,
                    dsti_v, srci_v, ev_v, rows_a, rows_b, rows_c,
                    acc_sh, y_sh, gs_a, gs_b, gs_c, fs_a, fs_b, fs_c):
    cid = lax.axis_index("c")
    sid = lax.axis_index("s")
    wid = sid * NC + cid

    def _zrow(i, carry):
        rows_a[i, 0:L] = jnp.zeros((L,), jnp.float32)
        rows_a[i, L:2 * L] = jnp.zeros((L,), jnp.float32)
        return carry

    lax.fori_loop(0, EC, _zrow, 0)
    pltpu.sync_copy(rows_a, acc_sh.at[pl.ds(sid * RPT, EC)])
    pltpu.sync_copy(rows_a.at[pl.ds(0, RPT - EC)],
                    acc_sh.at[pl.ds(sid * RPT + EC, RPT - EC)])
    pltpu.sync_copy(y_hbm.at[pl.ds(sid * RPT, RPT)],
                    y_sh.at[pl.ds(sid * RPT, RPT)])
    plsc.subcore_barrier()

    pltpu.sync_copy(dsti_hbm.at[wid], dsti_v)
    pltpu.sync_copy(srci_hbm.at[wid], srci_v)
    pltpu.sync_copy(ev_hbm.at[wid], ev_v)

    def _gather(c, buf, sem):
        return pltpu.make_async_copy(y_sh.at[dsti_v.at[c]], buf, sem)

    def _flush(c, buf, sem):
        return pltpu.make_async_copy(buf, acc_sh.at[srci_v.at[c]], sem)

    def _scale(buf, c):
        def _t(t, carry):
            ev16 = ev_v[c, pl.ds(t * L, L)]
            for k in range(L):
                i = t * L + k
                evb = _dyn_gather16(ev16, jnp.full((L,), k, jnp.int32))
                buf[i, 0:L] = buf[i, 0:L] * evb
                buf[i, L:2 * L] = buf[i, L:2 * L] * evb
            return carry

        lax.fori_loop(0, EC // L, _t, 0)

    bufs = (rows_a, rows_b, rows_c)
    gsems = (gs_a, gs_b, gs_c)
    fsems = (fs_a, fs_b, fs_c)

    _gather(0, rows_a, gs_a).start()
    _gather(1, rows_b, gs_b).start()

    def _pipe(jj, carry):
        for b_i in range(3):
            cc = 3 * jj + b_i
            _gather(cc, bufs[b_i], gsems[b_i]).wait()
            _scale(bufs[b_i], cc)
            _flush(cc, bufs[b_i], fsems[b_i]).start(add=True)
            nb = (b_i + 2) % 3

            @pl.when(cc > 0)
            def _w():
                _flush(cc - 1, bufs[nb], fsems[nb]).wait()

            @pl.when(cc + 2 < NG)
            def _g():
                _gather(cc + 2, bufs[nb], gsems[nb]).start()
        return carry

    lax.fori_loop(0, NG // 3, _pipe, 0)
    _gather(NG - 1, rows_a, gs_a).wait()
    _scale(rows_a, NG - 1)
    _flush(NG - 1, rows_a, fs_a).start(add=True)
    _flush(NG - 2, rows_c, fs_c).wait()
    _flush(NG - 1, rows_a, fs_a).wait()

    plsc.subcore_barrier()
    pltpu.sync_copy(
        acc_sh.at[pl.ds(sid * RPT, RPT)],
        out_hbm.at[cid, pl.ds(sid * RPT, RPT)],
    )


@functools.cache
def _make_sc_edgedot():
    mesh = plsc.VectorSubcoreMesh(core_axis_name="c", subcore_axis_name="s")
    return functools.partial(
        pl.kernel,
        out_type=jax.ShapeDtypeStruct((NW, NG, EC), jnp.float32),
        mesh=mesh,
        compiler_params=pltpu.CompilerParams(use_tc_tiling_on_sc=False),
        scratch_types=[
            pltpu.VMEM((NG, EC), jnp.int32),
            pltpu.VMEM((NG, EC), jnp.int32),
            pltpu.VMEM((NG, EC), jnp.float32),
            pltpu.VMEM((RPT, H), jnp.float32),
            pltpu.VMEM((RPT, H), jnp.float32),
            pltpu.VMEM((RPT, H), jnp.float32),
            pltpu.VMEM((EC, H), jnp.float32),
            pltpu.VMEM_SHARED((NP, H), jnp.float32),
            pltpu.SemaphoreType.DMA,
            pltpu.SemaphoreType.DMA,
        ],
    )(_sc_edgedot_body)


def _sc_edgedot_body(p_hbm, z_hbm, srci_hbm, dsti_hbm, out_hbm,
                     srci_v, dsti_v, dots_v, b0, b1, b2, b3,
                     hx_sh, sem_a, sem_b):
    cid = lax.axis_index("c")
    sid = lax.axis_index("s")
    wid = sid * NC + cid

    pltpu.sync_copy(p_hbm.at[0, pl.ds(sid * RPT, RPT)], b0)
    pltpu.sync_copy(p_hbm.at[1, pl.ds(sid * RPT, RPT)], b1)
    pltpu.sync_copy(z_hbm.at[pl.ds(sid * RPT, RPT)], b2)
    pltpu.sync_copy(srci_hbm.at[wid], srci_v)
    pltpu.sync_copy(dsti_hbm.at[wid], dsti_v)

    def _hxrow(r, carry):
        for h in range(2):
            sl = pl.ds(h * L, L)
            b0[r, sl] = jnp.maximum(b0[r, sl] + b1[r, sl] + b2[r, sl], 0.0)
        return carry

    lax.fori_loop(0, RPT, _hxrow, 0)
    pltpu.sync_copy(b0, hx_sh.at[pl.ds(sid * RPT, RPT)])
    plsc.subcore_barrier()

    iota16 = lax.iota(jnp.int32, L)

    def _g(c, idxv, buf, sem):
        return pltpu.make_async_copy(
            hx_sh.at[idxv.at[c]], buf.at[pl.ds(0, EC)], sem)

    def _start(c, sb, db, sem):
        _g(c, srci_v, sb, sem).start()
        _g(c, dsti_v, db, sem).start()

    def _wait(c, sb, db, sem):
        _g(c, srci_v, sb, sem).wait()
        _g(c, dsti_v, db, sem).wait()

    def _dot(sb, db, c):
        def _t(t, carry):
            acc = jnp.zeros((L,), jnp.float32)
            for k in range(L):
                i = t * L + k
                v = (sb[i, 0:L] * db[i, 0:L]
                     + sb[i, L:2 * L] * db[i, L:2 * L])
                for sh in (1, 2, 4, 8):
                    v = v + _dyn_gather16(v, jnp.bitwise_xor(iota16, sh))
                acc = jnp.where(iota16 == k, v, acc)
            dots_v[c, pl.ds(t * L, L)] = acc
            return carry

        lax.fori_loop(0, EC // L, _t, 0)

    _start(0, b0, b1, sem_a)

    def _pipe(jj, carry):
        c0 = 2 * jj
        _start(c0 + 1, b2, b3, sem_b)
        _wait(c0, b0, b1, sem_a)
        _dot(b0, b1, c0)
        _start(c0 + 2, b0, b1, sem_a)
        _wait(c0 + 1, b2, b3, sem_b)
        _dot(b2, b3, c0 + 1)
        return carry

    lax.fori_loop(0, (NG - 1) // 2, _pipe, 0)
    _wait(NG - 1, b0, b1, sem_a)
    _dot(b0, b1, NG - 1)

    pltpu.sync_copy(dots_v, out_hbm.at[wid])


def _loss_body(s_ref, c_ref, o_ref):
    s = s_ref[...]
    sx = 1.0 / (1.0 + jnp.exp(-s))
    sxc = jnp.clip(sx, 1e-12, 1.0 - 1e-7)
    w = jnp.where(sxc < 0.5, _SIMILAR_WEIGHT, 1.0)
    cf = c_ref[...].astype(jnp.float32)
    v = w * -(cf * jnp.log(sxc) + (1.0 - cf) * jnp.log(1.0 - sxc))
    o_ref[...] = (jnp.sum(v) / E).reshape(1, 1)


def _loss(logits2d, c2d):
    return pl.pallas_call(
        _loss_body,
        out_shape=jax.ShapeDtypeStruct((1, 1), jnp.float32),
    )(logits2d, c2d)


def kernel(X, edge_values, W_pass, b_pass, W_self, b_self, edge_index, C):
    src = edge_index[0]
    dst = edge_index[1]
    bias = (b_pass + b_self).reshape(1, H)

    Y, Z = _dense(X, W_pass, W_self, bias)

    src2 = src.reshape(NW, NG, EC)
    dst2 = dst.reshape(NW, NG, EC)
    ev2 = edge_values.reshape(NW, NG, EC)

    partials = _make_sc_segsum()(Y, dst2, src2, ev2)
    logits = _make_sc_edgedot()(partials, Z, src2, dst2)

    loss = _loss(logits.reshape(E // D, D), C.reshape(E // D, D))
    return loss[0, 0]


# speedup vs baseline: 7.1392x; 1.0249x over previous
import functools

import jax
import jax.numpy as jnp
from jax import lax
from jax.experimental import pallas as pl
from jax.experimental.pallas import tpu as pltpu
from jax.experimental.pallas import tpu_sc as plsc

N = 10000
E = 320000
D = 128
H = 32
_SIMILAR_WEIGHT = 1.0

NC = 2
NS = 16
L = 16
NW = NC * NS
EPW = E // NW
CH = 80
NCH = EPW // CH
SCH = 5
NG = NCH // SCH
EC = SCH * CH
NP = 10240
RPT = NP // NS

RPT2 = RPT // 2

ROW_BLK = 1000


def _dyn_gather16(vec, idx16):
    return lax.gather(
        vec,
        idx16[:, None],
        dimension_numbers=lax.GatherDimensionNumbers(
            offset_dims=(), collapsed_slice_dims=(0,), start_index_map=(0,)
        ),
        slice_sizes=(1,),
        mode=lax.GatherScatterMode.PROMISE_IN_BOUNDS,
    )


def _dense_body(x_ref, wp_ref, ws_ref, b_ref, y_ref, z_ref):
    x = x_ref[...]
    y_ref[...] = jnp.dot(x, wp_ref[...], preferred_element_type=jnp.float32)
    z_ref[...] = (
        jnp.dot(x, ws_ref[...], preferred_element_type=jnp.float32) + b_ref[...]
    )


def _dense(X, W_pass, W_self, bias):
    return pl.pallas_call(
        _dense_body,
        grid=(N // ROW_BLK,),
        in_specs=[
            pl.BlockSpec((ROW_BLK, D), lambda i: (i, 0)),
            pl.BlockSpec((D, H), lambda i: (0, 0)),
            pl.BlockSpec((D, H), lambda i: (0, 0)),
            pl.BlockSpec((1, H), lambda i: (0, 0)),
        ],
        out_specs=[
            pl.BlockSpec((ROW_BLK, H), lambda i: (i, 0)),
            pl.BlockSpec((ROW_BLK, H), lambda i: (i, 0)),
        ],
        out_shape=[
            jax.ShapeDtypeStruct((NP, H), jnp.float32),
            jax.ShapeDtypeStruct((NP, H), jnp.float32),
        ],
    )(X, W_pass, W_self, bias)


@functools.cache
def _make_sc_segsum():
    mesh = plsc.VectorSubcoreMesh(core_axis_name="c", subcore_axis_name="s")
    return functools.partial(
        pl.kernel,
        out_type=jax.ShapeDtypeStruct((NC, NP, H), jnp.float32),
        mesh=mesh,
        compiler_params=pltpu.CompilerParams(use_tc_tiling_on_sc=False),
        scratch_types=[
            pltpu.VMEM((NG, EC), jnp.int32),
            pltpu.VMEM((NG, EC), jnp.int32),
            pltpu.VMEM((NG, EC), jnp.float32),
            pltpu.VMEM((EC, H), jnp.float32),
            pltpu.VMEM((EC, H), jnp.float32),
            pltpu.VMEM((EC, H), jnp.float32),
            pltpu.VMEM_SHARED((NP, H), jnp.float32),
            pltpu.VMEM_SHARED((NP, H), jnp.float32),
            pltpu.SemaphoreType.DMA,
            pltpu.SemaphoreType.DMA,
            pltpu.SemaphoreType.DMA,
            pltpu.SemaphoreType.DMA,
            pltpu.SemaphoreType.DMA,
            pltpu.SemaphoreType.DMA,
        ],
    )(_sc_segsum_body)


def _sc_segsum_body(y_hbm, dsti_hbm, srci_hbm, ev_hbm, out_hbm,
                    dsti_v, srci_v, ev_v, rows_a, rows_b, rows_c,
                    acc_sh, y_sh, gs_a, gs_b, gs_c, fs_a, fs_b, fs_c):
    cid = lax.axis_index("c")
    sid = lax.axis_index("s")
    wid = sid * NC + cid

    def _zrow(i, carry):
        rows_a[i, 0:L] = jnp.zeros((L,), jnp.float32)
        rows_a[i, L:2 * L] = jnp.zeros((L,), jnp.float32)
        return carry

    lax.fori_loop(0, EC, _zrow, 0)
    pltpu.sync_copy(rows_a, acc_sh.at[pl.ds(sid * RPT, EC)])
    pltpu.sync_copy(rows_a.at[pl.ds(0, RPT - EC)],
                    acc_sh.at[pl.ds(sid * RPT + EC, RPT - EC)])
    pltpu.sync_copy(y_hbm.at[pl.ds(sid * RPT, RPT)],
                    y_sh.at[pl.ds(sid * RPT, RPT)])
    plsc.subcore_barrier()

    pltpu.sync_copy(dsti_hbm.at[wid], dsti_v)
    pltpu.sync_copy(srci_hbm.at[wid], srci_v)
    pltpu.sync_copy(ev_hbm.at[wid], ev_v)

    def _gather(c, buf, sem):
        return pltpu.make_async_copy(y_sh.at[dsti_v.at[c]], buf, sem)

    def _flush(c, buf, sem):
        return pltpu.make_async_copy(buf, acc_sh.at[srci_v.at[c]], sem)

    def _scale(buf, c):
        def _t(t, carry):
            ev16 = ev_v[c, pl.ds(t * L, L)]
            for k in range(L):
                i = t * L + k
                evb = _dyn_gather16(ev16, jnp.full((L,), k, jnp.int32))
                buf[i, 0:L] = buf[i, 0:L] * evb
                buf[i, L:2 * L] = buf[i, L:2 * L] * evb
            return carry

        lax.fori_loop(0, EC // L, _t, 0)

    bufs = (rows_a, rows_b, rows_c)
    gsems = (gs_a, gs_b, gs_c)
    fsems = (fs_a, fs_b, fs_c)

    _gather(0, rows_a, gs_a).start()
    _gather(1, rows_b, gs_b).start()

    def _pipe(jj, carry):
        for b_i in range(3):
            cc = 3 * jj + b_i
            _gather(cc, bufs[b_i], gsems[b_i]).wait()
            _scale(bufs[b_i], cc)
            _flush(cc, bufs[b_i], fsems[b_i]).start(add=True)
            nb = (b_i + 2) % 3

            @pl.when(cc > 0)
            def _w():
                _flush(cc - 1, bufs[nb], fsems[nb]).wait()

            @pl.when(cc + 2 < NG)
            def _g():
                _gather(cc + 2, bufs[nb], gsems[nb]).start()
        return carry

    lax.fori_loop(0, NG // 3, _pipe, 0)
    _gather(NG - 1, rows_a, gs_a).wait()
    _scale(rows_a, NG - 1)
    _flush(NG - 1, rows_a, fs_a).start(add=True)
    _flush(NG - 2, rows_c, fs_c).wait()
    _flush(NG - 1, rows_a, fs_a).wait()

    plsc.subcore_barrier()
    pltpu.sync_copy(
        acc_sh.at[pl.ds(sid * RPT, RPT)],
        out_hbm.at[cid, pl.ds(sid * RPT, RPT)],
    )


@functools.cache
def _make_sc_edgedot():
    mesh = plsc.VectorSubcoreMesh(core_axis_name="c", subcore_axis_name="s")
    return functools.partial(
        pl.kernel,
        out_type=jax.ShapeDtypeStruct((NW, NG, EC), jnp.float32),
        mesh=mesh,
        compiler_params=pltpu.CompilerParams(use_tc_tiling_on_sc=False),
        scratch_types=[
            pltpu.VMEM((NG, EC), jnp.int32),
            pltpu.VMEM((NG, EC), jnp.int32),
            pltpu.VMEM((NG, EC), jnp.float32),
            pltpu.VMEM((RPT2, H), jnp.float32),
            pltpu.VMEM((RPT2, H), jnp.float32),
            pltpu.VMEM((RPT2, H), jnp.float32),
            pltpu.VMEM((RPT2, L), jnp.int32),
            pltpu.VMEM((EC, L), jnp.int32),
            pltpu.VMEM((EC, L), jnp.int32),
            pltpu.VMEM((EC, L), jnp.int32),
            pltpu.VMEM((EC, L), jnp.int32),
            pltpu.VMEM_SHARED((NP, L), jnp.int32),
            pltpu.SemaphoreType.DMA,
            pltpu.SemaphoreType.DMA,
        ],
    )(_sc_edgedot_body)


def _sc_edgedot_body(p_hbm, z_hbm, srci_hbm, dsti_hbm, out_hbm,
                     srci_v, dsti_v, dots_v, s0_v, s1_v, sz_v, hb_v,
                     b0, b1, b2, b3, hx_sh, sem_a, sem_b):
    cid = lax.axis_index("c")
    sid = lax.axis_index("s")
    wid = sid * NC + cid

    pltpu.sync_copy(srci_hbm.at[wid], srci_v)
    pltpu.sync_copy(dsti_hbm.at[wid], dsti_v)
    for half in range(2):
        base = sid * RPT + half * RPT2
        pltpu.sync_copy(p_hbm.at[0, pl.ds(base, RPT2)], s0_v)
        pltpu.sync_copy(p_hbm.at[1, pl.ds(base, RPT2)], s1_v)
        pltpu.sync_copy(z_hbm.at[pl.ds(base, RPT2)], sz_v)

        def _hxrow(r, carry):
            h0 = jnp.maximum(s0_v[r, 0:L] + s1_v[r, 0:L] + sz_v[r, 0:L], 0.0)
            h1 = jnp.maximum(
                s0_v[r, L:2 * L] + s1_v[r, L:2 * L] + sz_v[r, L:2 * L], 0.0)
            u0 = lax.bitcast_convert_type(h0, jnp.int32)
            u0 = u0 + 0x7FFF + (lax.shift_right_logical(u0, 16) & 1)
            u1 = lax.bitcast_convert_type(h1, jnp.int32)
            u1 = u1 + 0x7FFF + (lax.shift_right_logical(u1, 16) & 1)
            hb_v[r, 0:L] = jnp.bitwise_or(
                jnp.bitwise_and(u1, jnp.int32(-65536)),
                lax.shift_right_logical(u0, 16),
            )
            return carry

        lax.fori_loop(0, RPT2, _hxrow, 0)
        pltpu.sync_copy(hb_v, hx_sh.at[pl.ds(base, RPT2)])
    plsc.subcore_barrier()

    iota16 = lax.iota(jnp.int32, L)

    def _g(c, idxv, buf, sem):
        return pltpu.make_async_copy(
            hx_sh.at[idxv.at[c]], buf.at[pl.ds(0, EC)], sem)

    def _start(c, sb, db, sem):
        _g(c, srci_v, sb, sem).start()
        _g(c, dsti_v, db, sem).start()

    def _wait(c, sb, db, sem):
        _g(c, srci_v, sb, sem).wait()
        _g(c, dsti_v, db, sem).wait()

    def _dot(sb, db, c):
        def _t(t, carry):
            vs = []
            mhi = jnp.int32(-65536)
            for k in range(L):
                i = t * L + k
                ws = sb[i, 0:L]
                wd = db[i, 0:L]
                a0 = lax.bitcast_convert_type(
                    lax.shift_left(ws, 16), jnp.float32)
                a1 = lax.bitcast_convert_type(jnp.bitwise_and(ws, mhi),
                                              jnp.float32)
                c0 = lax.bitcast_convert_type(
                    lax.shift_left(wd, 16), jnp.float32)
                c1 = lax.bitcast_convert_type(jnp.bitwise_and(wd, mhi),
                                              jnp.float32)
                vs.append(a0 * c0 + a1 * c1)
            for sh in (1, 2, 4, 8):
                nxt = []
                for m in range(0, len(vs), 2):
                    ta = vs[m] + _dyn_gather16(vs[m], iota16 ^ sh)
                    tb = vs[m + 1] + _dyn_gather16(vs[m + 1], iota16 ^ sh)
                    nxt.append(jnp.where((iota16 & sh) == 0, ta, tb))
                vs = nxt
            dots_v[c, pl.ds(t * L, L)] = vs[0]
            return carry

        lax.fori_loop(0, EC // L, _t, 0)

    _start(0, b0, b1, sem_a)

    def _pipe(jj, carry):
        c0 = 2 * jj
        _start(c0 + 1, b2, b3, sem_b)
        _wait(c0, b0, b1, sem_a)
        _dot(b0, b1, c0)
        _start(c0 + 2, b0, b1, sem_a)
        _wait(c0 + 1, b2, b3, sem_b)
        _dot(b2, b3, c0 + 1)
        return carry

    lax.fori_loop(0, (NG - 1) // 2, _pipe, 0)
    _wait(NG - 1, b0, b1, sem_a)
    _dot(b0, b1, NG - 1)

    pltpu.sync_copy(dots_v, out_hbm.at[wid])


def _loss_body(s_ref, c_ref, o_ref):
    s = s_ref[...]
    sx = 1.0 / (1.0 + jnp.exp(-s))
    sxc = jnp.clip(sx, 1e-12, 1.0 - 1e-7)
    w = jnp.where(sxc < 0.5, _SIMILAR_WEIGHT, 1.0)
    cf = c_ref[...].astype(jnp.float32)
    v = w * -(cf * jnp.log(sxc) + (1.0 - cf) * jnp.log(1.0 - sxc))
    o_ref[...] = (jnp.sum(v) / E).reshape(1, 1)


def _loss(logits2d, c2d):
    return pl.pallas_call(
        _loss_body,
        out_shape=jax.ShapeDtypeStruct((1, 1), jnp.float32),
    )(logits2d, c2d)


def kernel(X, edge_values, W_pass, b_pass, W_self, b_self, edge_index, C):
    src = edge_index[0]
    dst = edge_index[1]
    bias = (b_pass + b_self).reshape(1, H)

    Y, Z = _dense(X, W_pass, W_self, bias)

    src2 = src.reshape(NW, NG, EC)
    dst2 = dst.reshape(NW, NG, EC)
    ev2 = edge_values.reshape(NW, NG, EC)

    partials = _make_sc_segsum()(Y, dst2, src2, ev2)
    logits = _make_sc_edgedot()(partials, Z, src2, dst2)

    loss = _loss(logits.reshape(E // D, D), C.reshape(E // D, D))
    return loss[0, 0]


# speedup vs baseline: 7.3369x; 1.0277x over previous
import functools

import jax
import jax.numpy as jnp
from jax import lax
from jax.experimental import pallas as pl
from jax.experimental.pallas import tpu as pltpu
from jax.experimental.pallas import tpu_sc as plsc

N = 10000
E = 320000
D = 128
H = 32
_SIMILAR_WEIGHT = 1.0

NC = 2
NS = 16
L = 16
NW = NC * NS
EPW = E // NW
CH = 80
NCH = EPW // CH
SCH = 5
NG = NCH // SCH
EC = SCH * CH
NP = 10240
RPT = NP // NS

RPT2 = RPT // 2

ROW_BLK = 1000


def _dyn_gather16(vec, idx16):
    return lax.gather(
        vec,
        idx16[:, None],
        dimension_numbers=lax.GatherDimensionNumbers(
            offset_dims=(), collapsed_slice_dims=(0,), start_index_map=(0,)
        ),
        slice_sizes=(1,),
        mode=lax.GatherScatterMode.PROMISE_IN_BOUNDS,
    )


def _dense_body(x_ref, wp_ref, ws_ref, b_ref, y_ref, z_ref):
    x = x_ref[...]
    y_ref[...] = jnp.dot(x, wp_ref[...], preferred_element_type=jnp.float32)
    z_ref[...] = (
        jnp.dot(x, ws_ref[...], preferred_element_type=jnp.float32) + b_ref[...]
    )


def _dense(X, W_pass, W_self, bias):
    return pl.pallas_call(
        _dense_body,
        grid=(N // ROW_BLK,),
        in_specs=[
            pl.BlockSpec((ROW_BLK, D), lambda i: (i, 0)),
            pl.BlockSpec((D, H), lambda i: (0, 0)),
            pl.BlockSpec((D, H), lambda i: (0, 0)),
            pl.BlockSpec((1, H), lambda i: (0, 0)),
        ],
        out_specs=[
            pl.BlockSpec((ROW_BLK, H), lambda i: (i, 0)),
            pl.BlockSpec((ROW_BLK, H), lambda i: (i, 0)),
        ],
        out_shape=[
            jax.ShapeDtypeStruct((NP, H), jnp.float32),
            jax.ShapeDtypeStruct((NP, H), jnp.float32),
        ],
    )(X, W_pass, W_self, bias)


@functools.cache
def _make_sc_segsum():
    mesh = plsc.VectorSubcoreMesh(core_axis_name="c", subcore_axis_name="s")
    return functools.partial(
        pl.kernel,
        out_type=jax.ShapeDtypeStruct((NC, NP, H), jnp.float32),
        mesh=mesh,
        compiler_params=pltpu.CompilerParams(use_tc_tiling_on_sc=False),
        scratch_types=[
            pltpu.VMEM((NG, EC), jnp.int32),
            pltpu.VMEM((NG, EC), jnp.int32),
            pltpu.VMEM((NG, EC), jnp.float32),
            pltpu.VMEM((EC, H), jnp.float32),
            pltpu.VMEM((EC, H), jnp.float32),
            pltpu.VMEM((EC, H), jnp.float32),
            pltpu.VMEM_SHARED((NP, H), jnp.float32),
            pltpu.VMEM_SHARED((NP, H), jnp.float32),
            pltpu.SemaphoreType.DMA,
            pltpu.SemaphoreType.DMA,
            pltpu.SemaphoreType.DMA,
            pltpu.SemaphoreType.DMA,
            pltpu.SemaphoreType.DMA,
            pltpu.SemaphoreType.DMA,
        ],
    )(_sc_segsum_body)


def _sc_segsum_body(y_hbm, dsti_hbm, srci_hbm, ev_hbm, out_hbm,
                    dsti_v, srci_v, ev_v, rows_a, rows_b, rows_c,
                    acc_sh, y_sh, gs_a, gs_b, gs_c, fs_a, fs_b, fs_c):
    cid = lax.axis_index("c")
    sid = lax.axis_index("s")
    wid = sid * NC + cid

    def _zrow(i, carry):
        rows_a[i, 0:L] = jnp.zeros((L,), jnp.float32)
        rows_a[i, L:2 * L] = jnp.zeros((L,), jnp.float32)
        return carry

    lax.fori_loop(0, EC, _zrow, 0)
    pltpu.sync_copy(rows_a, acc_sh.at[pl.ds(sid * RPT, EC)])
    pltpu.sync_copy(rows_a.at[pl.ds(0, RPT - EC)],
                    acc_sh.at[pl.ds(sid * RPT + EC, RPT - EC)])
    pltpu.sync_copy(y_hbm.at[pl.ds(sid * RPT, RPT)],
                    y_sh.at[pl.ds(sid * RPT, RPT)])
    plsc.subcore_barrier()

    pltpu.sync_copy(dsti_hbm.at[wid], dsti_v)
    pltpu.sync_copy(srci_hbm.at[wid], srci_v)
    pltpu.sync_copy(ev_hbm.at[wid], ev_v)

    def _gather(c, buf, sem):
        return pltpu.make_async_copy(y_sh.at[dsti_v.at[c]], buf, sem)

    def _flush(c, buf, sem):
        return pltpu.make_async_copy(buf, acc_sh.at[srci_v.at[c]], sem)

    def _scale(buf, c):
        def _t(t, carry):
            ev16 = ev_v[c, pl.ds(t * L, L)]
            for k in range(L):
                i = t * L + k
                evb = _dyn_gather16(ev16, jnp.full((L,), k, jnp.int32))
                buf[i, 0:L] = buf[i, 0:L] * evb
                buf[i, L:2 * L] = buf[i, L:2 * L] * evb
            return carry

        lax.fori_loop(0, EC // L, _t, 0)

    bufs = (rows_a, rows_b, rows_c)
    gsems = (gs_a, gs_b, gs_c)
    fsems = (fs_a, fs_b, fs_c)

    _gather(0, rows_a, gs_a).start()
    _gather(1, rows_b, gs_b).start()

    def _pipe(jj, carry):
        for b_i in range(3):
            cc = 3 * jj + b_i
            _gather(cc, bufs[b_i], gsems[b_i]).wait()
            _scale(bufs[b_i], cc)
            _flush(cc, bufs[b_i], fsems[b_i]).start(add=True)
            nb = (b_i + 2) % 3

            @pl.when(cc > 0)
            def _w():
                _flush(cc - 1, bufs[nb], fsems[nb]).wait()

            @pl.when(cc + 2 < NG)
            def _g():
                _gather(cc + 2, bufs[nb], gsems[nb]).start()
        return carry

    lax.fori_loop(0, NG // 3, _pipe, 0)
    _gather(NG - 1, rows_a, gs_a).wait()
    _scale(rows_a, NG - 1)
    _flush(NG - 1, rows_a, fs_a).start(add=True)
    _flush(NG - 2, rows_c, fs_c).wait()
    _flush(NG - 1, rows_a, fs_a).wait()

    plsc.subcore_barrier()
    pltpu.sync_copy(
        acc_sh.at[pl.ds(sid * RPT, RPT)],
        out_hbm.at[cid, pl.ds(sid * RPT, RPT)],
    )


@functools.cache
def _make_sc_edgedot():
    mesh = plsc.VectorSubcoreMesh(core_axis_name="c", subcore_axis_name="s")
    return functools.partial(
        pl.kernel,
        out_type=jax.ShapeDtypeStruct((NW, NG, EC), jnp.float32),
        mesh=mesh,
        compiler_params=pltpu.CompilerParams(use_tc_tiling_on_sc=False),
        scratch_types=[
            pltpu.VMEM((NG, EC), jnp.int32),
            pltpu.VMEM((NG, EC), jnp.int32),
            pltpu.VMEM((NG, EC), jnp.float32),
            pltpu.VMEM((RPT2, H), jnp.float32),
            pltpu.VMEM((RPT2, H), jnp.float32),
            pltpu.VMEM((RPT2, H), jnp.float32),
            pltpu.VMEM((RPT2, L), jnp.int32),
            pltpu.VMEM((EC, L), jnp.int32),
            pltpu.VMEM((EC, L), jnp.int32),
            pltpu.VMEM((EC, L), jnp.int32),
            pltpu.VMEM((EC, L), jnp.int32),
            pltpu.VMEM_SHARED((NP, L), jnp.int32),
            pltpu.SemaphoreType.DMA,
            pltpu.SemaphoreType.DMA,
        ],
    )(_sc_edgedot_body)


def _sc_edgedot_body(p_hbm, z_hbm, srci_hbm, dsti_hbm, out_hbm,
                     srci_v, dsti_v, dots_v, s0_v, s1_v, sz_v, hb_v,
                     b0, b1, b2, b3, hx_sh, sem_a, sem_b):
    cid = lax.axis_index("c")
    sid = lax.axis_index("s")
    wid = sid * NC + cid

    i1 = pltpu.make_async_copy(srci_hbm.at[wid], srci_v, sem_a)
    i2 = pltpu.make_async_copy(dsti_hbm.at[wid], dsti_v, sem_a)
    i1.start()
    i2.start()
    for half in range(2):
        base = sid * RPT + half * RPT2
        p1c = pltpu.make_async_copy(p_hbm.at[0, pl.ds(base, RPT2)], s0_v,
                                    sem_b)
        p2c = pltpu.make_async_copy(p_hbm.at[1, pl.ds(base, RPT2)], s1_v,
                                    sem_b)
        p3c = pltpu.make_async_copy(z_hbm.at[pl.ds(base, RPT2)], sz_v, sem_b)
        p1c.start()
        p2c.start()
        p3c.start()
        p1c.wait()
        p2c.wait()
        p3c.wait()

        def _hxrow(r, carry):
            h0 = jnp.maximum(s0_v[r, 0:L] + s1_v[r, 0:L] + sz_v[r, 0:L], 0.0)
            h1 = jnp.maximum(
                s0_v[r, L:2 * L] + s1_v[r, L:2 * L] + sz_v[r, L:2 * L], 0.0)
            u0 = lax.bitcast_convert_type(h0, jnp.int32)
            u0 = u0 + 0x7FFF + (lax.shift_right_logical(u0, 16) & 1)
            u1 = lax.bitcast_convert_type(h1, jnp.int32)
            u1 = u1 + 0x7FFF + (lax.shift_right_logical(u1, 16) & 1)
            hb_v[r, 0:L] = jnp.bitwise_or(
                jnp.bitwise_and(u1, jnp.int32(-65536)),
                lax.shift_right_logical(u0, 16),
            )
            return carry

        lax.fori_loop(0, RPT2, _hxrow, 0)
        pltpu.sync_copy(hb_v, hx_sh.at[pl.ds(base, RPT2)])
    i1.wait()
    i2.wait()
    plsc.subcore_barrier()

    iota16 = lax.iota(jnp.int32, L)

    def _g(c, idxv, buf, sem):
        return pltpu.make_async_copy(
            hx_sh.at[idxv.at[c]], buf.at[pl.ds(0, EC)], sem)

    def _start(c, sb, db, sem):
        _g(c, srci_v, sb, sem).start()
        _g(c, dsti_v, db, sem).start()

    def _wait(c, sb, db, sem):
        _g(c, srci_v, sb, sem).wait()
        _g(c, dsti_v, db, sem).wait()

    def _dot(sb, db, c):
        def _t(t, carry):
            vs = []
            mhi = jnp.int32(-65536)
            for k in range(L):
                i = t * L + k
                ws = sb[i, 0:L]
                wd = db[i, 0:L]
                a0 = lax.bitcast_convert_type(
                    lax.shift_left(ws, 16), jnp.float32)
                a1 = lax.bitcast_convert_type(jnp.bitwise_and(ws, mhi),
                                              jnp.float32)
                c0 = lax.bitcast_convert_type(
                    lax.shift_left(wd, 16), jnp.float32)
                c1 = lax.bitcast_convert_type(jnp.bitwise_and(wd, mhi),
                                              jnp.float32)
                vs.append(a0 * c0 + a1 * c1)
            for sh in (1, 2, 4, 8):
                nxt = []
                for m in range(0, len(vs), 2):
                    ta = vs[m] + _dyn_gather16(vs[m], iota16 ^ sh)
                    tb = vs[m + 1] + _dyn_gather16(vs[m + 1], iota16 ^ sh)
                    nxt.append(jnp.where((iota16 & sh) == 0, ta, tb))
                vs = nxt
            dots_v[c, pl.ds(t * L, L)] = vs[0]
            return carry

        lax.fori_loop(0, EC // L, _t, 0)

    _start(0, b0, b1, sem_a)

    def _pipe(jj, carry):
        c0 = 2 * jj
        _start(c0 + 1, b2, b3, sem_b)
        _wait(c0, b0, b1, sem_a)
        _dot(b0, b1, c0)
        _start(c0 + 2, b0, b1, sem_a)
        _wait(c0 + 1, b2, b3, sem_b)
        _dot(b2, b3, c0 + 1)
        return carry

    lax.fori_loop(0, (NG - 1) // 2, _pipe, 0)
    _wait(NG - 1, b0, b1, sem_a)
    _dot(b0, b1, NG - 1)

    pltpu.sync_copy(dots_v, out_hbm.at[wid])


def _loss_body(s_ref, c_ref, o_ref):
    s = s_ref[...]
    sx = 1.0 / (1.0 + jnp.exp(-s))
    sxc = jnp.clip(sx, 1e-12, 1.0 - 1e-7)
    w = jnp.where(sxc < 0.5, _SIMILAR_WEIGHT, 1.0)
    cf = c_ref[...].astype(jnp.float32)
    v = w * -(cf * jnp.log(sxc) + (1.0 - cf) * jnp.log(1.0 - sxc))
    o_ref[...] = (jnp.sum(v) / E).reshape(1, 1)


def _loss(logits2d, c2d):
    return pl.pallas_call(
        _loss_body,
        out_shape=jax.ShapeDtypeStruct((1, 1), jnp.float32),
    )(logits2d, c2d)


def kernel(X, edge_values, W_pass, b_pass, W_self, b_self, edge_index, C):
    src = edge_index[0]
    dst = edge_index[1]
    bias = (b_pass + b_self).reshape(1, H)

    Y, Z = _dense(X, W_pass, W_self, bias)

    src2 = src.reshape(NW, NG, EC)
    dst2 = dst.reshape(NW, NG, EC)
    ev2 = edge_values.reshape(NW, NG, EC)

    partials = _make_sc_segsum()(Y, dst2, src2, ev2)
    logits = _make_sc_edgedot()(partials, Z, src2, dst2)

    loss = _loss(logits.reshape(E // D, D), C.reshape(E // D, D))
    return loss[0, 0]
